# baseline (speedup 1.0000x reference)
import functools

import numpy as _np

import jax
import jax.numpy as jnp
from jax import lax
from jax.experimental import pallas as pl
from jax.experimental.pallas import tpu as pltpu
from jax.experimental.pallas import tpu_sc as plsc

N = 10000
E = 320000
D = 128
H = 128
C = 40

NC = 2
NS = 16
NW = NC * NS

K = 128
EPW = 10240
NCHUNK = EPW // K
E_PAD = NW * EPW
ACC_ROWS = 10240
ROWS_PER_SUB = ACC_ROWS // NS

MB = 2000
NBLK = N // MB


NROW = 2
NIDX = 4
RPW = E // NW
NFULL = RPW // K
NTAIL = RPW - NFULL * K
ZR = 32


def _idx_load(edge_hbm, pad_hbm, kind, w, g, slot, sem):
    base = kind * E + w * RPW
    prow = kind * NW * 2 + w * 2

    @pl.when(g < NFULL)
    def _():
        pltpu.async_copy(edge_hbm.at[pl.ds(base + g * K, K)], slot, sem)

    @pl.when(g == NFULL)
    def _():
        pltpu.async_copy(edge_hbm.at[pl.ds(base + NFULL * K, NTAIL)],
                         slot.at[pl.ds(0, NTAIL)], sem)
        pltpu.async_copy(pad_hbm.at[prow, pl.ds(NTAIL, K - NTAIL)],
                         slot.at[pl.ds(NTAIL, K - NTAIL)], sem)

    @pl.when(g == NFULL + 1)
    def _():
        pltpu.async_copy(pad_hbm.at[prow + 1], slot, sem)


def _segsum_body(h_hbm, edge_hbm, pad_hbm, out_hbm,
                 si0, si1, si2, si3, di0, di1, di2, di3, rows0, rows1,
                 zeros_v, acc,
                 gs0, gs1, is0, is1, is2, is3, ds0, ds1, ds2, ds3):
    rows = (rows0, rows1)
    gsem = (gs0, gs1)
    src_s = (si0, si1, si2, si3)
    isem = (is0, is1, is2, is3)
    dst_s = (di0, di1, di2, di3)
    dsem = (ds0, ds1, ds2, ds3)
    c = lax.axis_index("c")
    s = lax.axis_index("s")
    w = c * NS + s

    for t in range(NIDX):
        pltpu.async_copy(edge_hbm.at[pl.ds(w * RPW + t * K, K)],
                         src_s[t], isem[t])
        pltpu.async_copy(edge_hbm.at[pl.ds(E + w * RPW + t * K, K)],
                         dst_s[t], dsem[t])
    for b in range(NROW):
        pltpu.make_async_copy(edge_hbm.at[pl.ds(0, K)], src_s[b],
                              isem[b]).wait()
        pltpu.async_copy(h_hbm.at[src_s[b]], rows[b], gsem[b])

    for r in range(ZR):
        for j in range(D // 16):
            zeros_v[r, pl.ds(j * 16, 16)] = jnp.zeros((16,), jnp.float32)
    for j in range(ROWS_PER_SUB // ZR):
        r0 = s * ROWS_PER_SUB + j * ZR
        pltpu.sync_copy(zeros_v, acc.at[pl.ds(r0, ZR)])
    plsc.subcore_barrier()

    def chunkgrp(i, carry):
        g0 = i * NIDX
        for b in range(NIDX):
            g = g0 + b
            rb = b % NROW
            pltpu.make_async_copy(h_hbm.at[src_s[b]], rows[rb],
                                  gsem[rb]).wait()

            @pl.when(g + NIDX < NCHUNK)
            def _():
                _idx_load(edge_hbm, pad_hbm, 0, w, g + NIDX, src_s[b],
                          isem[b])

            pltpu.make_async_copy(edge_hbm.at[pl.ds(0, K)], dst_s[b],
                                  dsem[b]).wait()
            pltpu.sync_copy(rows[rb], acc.at[dst_s[b]], add=True)

            @pl.when(g + NIDX < NCHUNK)
            def _():
                _idx_load(edge_hbm, pad_hbm, 1, w, g + NIDX, dst_s[b],
                          dsem[b])

            @pl.when(g + NROW < NCHUNK)
            def _():
                tn = (b + NROW) % NIDX
                pltpu.make_async_copy(edge_hbm.at[pl.ds(0, K)],
                                      src_s[tn], isem[tn]).wait()
                pltpu.async_copy(h_hbm.at[src_s[tn]], rows[rb], gsem[rb])
        return carry

    lax.fori_loop(0, NCHUNK // NIDX, chunkgrp, 0)
    plsc.subcore_barrier()

    for j in range(ROWS_PER_SUB // K):
        r0 = s * ROWS_PER_SUB + j * K
        pltpu.sync_copy(acc.at[pl.ds(r0, K)], out_hbm.at[c, pl.ds(r0, K)])


def _make_segsum():
    mesh = plsc.VectorSubcoreMesh(core_axis_name="c", subcore_axis_name="s",
                                  num_cores=NC, num_subcores=NS)
    return pl.kernel(
        _segsum_body,
        out_type=jax.ShapeDtypeStruct((NC, ACC_ROWS, D), jnp.float32),
        mesh=mesh,
        scratch_types=[
            pltpu.VMEM((K,), jnp.int32),
            pltpu.VMEM((K,), jnp.int32),
            pltpu.VMEM((K,), jnp.int32),
            pltpu.VMEM((K,), jnp.int32),
            pltpu.VMEM((K,), jnp.int32),
            pltpu.VMEM((K,), jnp.int32),
            pltpu.VMEM((K,), jnp.int32),
            pltpu.VMEM((K,), jnp.int32),
            pltpu.VMEM((K, D), jnp.float32),
            pltpu.VMEM((K, D), jnp.float32),
            pltpu.VMEM((ZR, D), jnp.float32),
            pltpu.VMEM_SHARED((ACC_ROWS, D), jnp.float32),
            pltpu.SemaphoreType.DMA,
            pltpu.SemaphoreType.DMA,
            pltpu.SemaphoreType.DMA,
            pltpu.SemaphoreType.DMA,
            pltpu.SemaphoreType.DMA,
            pltpu.SemaphoreType.DMA,
            pltpu.SemaphoreType.DMA,
            pltpu.SemaphoreType.DMA,
            pltpu.SemaphoreType.DMA,
            pltpu.SemaphoreType.DMA,
        ],
    )


def _dense1_body(p_ref, x_ref, wrel_ref, wroot_ref, b_ref, o_ref):
    agg = p_ref[0] + p_ref[1]
    acc = jnp.dot(agg, wrel_ref[...], preferred_element_type=jnp.float32,
                  precision=lax.Precision.DEFAULT)
    acc += jnp.dot(x_ref[...], wroot_ref[...], preferred_element_type=jnp.float32,
                   precision=lax.Precision.DEFAULT)
    acc += b_ref[...]
    o_ref[...] = jnp.maximum(acc, 0.0)


def _dense1(p, x, W_rel, W_root, b):
    return pl.pallas_call(
        _dense1_body,
        grid=(NBLK,),
        in_specs=[
            pl.BlockSpec((NC, MB, D), lambda i: (0, i, 0)),
            pl.BlockSpec((MB, D), lambda i: (i, 0)),
            pl.BlockSpec((D, H), lambda i: (0, 0)),
            pl.BlockSpec((D, H), lambda i: (0, 0)),
            pl.BlockSpec((1, H), lambda i: (0, 0)),
        ],
        out_specs=pl.BlockSpec((MB, H), lambda i: (i, 0)),
        out_shape=jax.ShapeDtypeStruct((N, H), jnp.float32),
    )(p, x, W_rel, W_root, b.reshape(1, H))


def _dense2_body(p_ref, h_ref, wrel_ref, wroot_ref, b_ref, wc_ref, bc_ref, o_ref):
    agg = p_ref[0] + p_ref[1]
    t = jnp.dot(agg, wrel_ref[...], preferred_element_type=jnp.float32,
                precision=lax.Precision.DEFAULT)
    t += jnp.dot(h_ref[...], wroot_ref[...], preferred_element_type=jnp.float32,
                 precision=lax.Precision.DEFAULT)
    t += b_ref[...]
    o_ref[...] = jnp.dot(t, wc_ref[...], preferred_element_type=jnp.float32,
                         precision=lax.Precision.DEFAULT) + bc_ref[...]


def _dense2(p, h, W_rel, W_root, b, Wc, bc):
    return pl.pallas_call(
        _dense2_body,
        grid=(NBLK,),
        in_specs=[
            pl.BlockSpec((NC, MB, H), lambda i: (0, i, 0)),
            pl.BlockSpec((MB, H), lambda i: (i, 0)),
            pl.BlockSpec((H, H), lambda i: (0, 0)),
            pl.BlockSpec((H, H), lambda i: (0, 0)),
            pl.BlockSpec((1, H), lambda i: (0, 0)),
            pl.BlockSpec((H, C), lambda i: (0, 0)),
            pl.BlockSpec((1, C), lambda i: (0, 0)),
        ],
        out_specs=pl.BlockSpec((MB, C), lambda i: (i, 0)),
        out_shape=jax.ShapeDtypeStruct((N, C), jnp.float32),
    )(p, h, W_rel, W_root, b.reshape(1, H), Wc, bc.reshape(1, C))


_PAD_IDX = _np.arange(NW * 2 * K).reshape(NW * 2, K)
_PAD_NP = _np.concatenate([(_PAD_IDX * 41) % N,
                           N + _PAD_IDX % (ACC_ROWS - N)]).astype(_np.int32)


def kernel(x, edge_index, W1_rel, b1_rel, W1_root, W2_rel, b2_rel, W2_root, Wc, bc):
    pad_blk = jnp.asarray(_PAD_NP)
    edge_flat = edge_index.reshape(2 * E)
    segsum = _make_segsum()
    p1 = segsum(x, edge_flat, pad_blk)
    h1 = _dense1(p1, x, W1_rel, W1_root, b1_rel)
    p2 = segsum(h1, edge_flat, pad_blk)
    return _dense2(p2, h1, W2_rel, W2_root, b2_rel, Wc, bc)


# speedup vs baseline: 1.0031x; 1.0031x over previous
import functools

import numpy as _np

import jax
import jax.numpy as jnp
from jax import lax
from jax.experimental import pallas as pl
from jax.experimental.pallas import tpu as pltpu
from jax.experimental.pallas import tpu_sc as plsc

N = 10000
E = 320000
D = 128
H = 128
C = 40

NC = 2
NS = 16
NW = NC * NS

K = 128
EPW = 10240
NCHUNK = EPW // K
E_PAD = NW * EPW
ACC_ROWS = 10240
ROWS_PER_SUB = ACC_ROWS // NS

MB = 2000
NBLK = N // MB


NROW = 2
NIDX = 4
RPW = E // NW
NFULL = RPW // K
NTAIL = RPW - NFULL * K
ZR = 64


def _idx_load(edge_hbm, pad_hbm, kind, w, g, slot, sem):
    base = kind * E + w * RPW
    prow = kind * NW * 2 + w * 2

    @pl.when(g < NFULL)
    def _():
        pltpu.async_copy(edge_hbm.at[pl.ds(base + g * K, K)], slot, sem)

    @pl.when(g == NFULL)
    def _():
        pltpu.async_copy(edge_hbm.at[pl.ds(base + NFULL * K, NTAIL)],
                         slot.at[pl.ds(0, NTAIL)], sem)
        pltpu.async_copy(pad_hbm.at[prow, pl.ds(NTAIL, K - NTAIL)],
                         slot.at[pl.ds(NTAIL, K - NTAIL)], sem)

    @pl.when(g == NFULL + 1)
    def _():
        pltpu.async_copy(pad_hbm.at[prow + 1], slot, sem)


def _segsum_body(h_hbm, edge_hbm, pad_hbm, out_hbm,
                 si0, si1, si2, si3, di0, di1, di2, di3, rows0, rows1,
                 zeros_v, acc,
                 gs0, gs1, is0, is1, is2, is3, ds0, ds1, ds2, ds3):
    rows = (rows0, rows1)
    gsem = (gs0, gs1)
    src_s = (si0, si1, si2, si3)
    isem = (is0, is1, is2, is3)
    dst_s = (di0, di1, di2, di3)
    dsem = (ds0, ds1, ds2, ds3)
    c = lax.axis_index("c")
    s = lax.axis_index("s")
    w = c * NS + s

    for t in range(NIDX):
        pltpu.async_copy(edge_hbm.at[pl.ds(w * RPW + t * K, K)],
                         src_s[t], isem[t])
        pltpu.async_copy(edge_hbm.at[pl.ds(E + w * RPW + t * K, K)],
                         dst_s[t], dsem[t])
    for b in range(NROW):
        pltpu.make_async_copy(edge_hbm.at[pl.ds(0, K)], src_s[b],
                              isem[b]).wait()
        pltpu.async_copy(h_hbm.at[src_s[b]], rows[b], gsem[b])

    for r in range(ZR):
        for j in range(D // 16):
            zeros_v[r, pl.ds(j * 16, 16)] = jnp.zeros((16,), jnp.float32)
    for j in range(ROWS_PER_SUB // ZR):
        r0 = s * ROWS_PER_SUB + j * ZR
        pltpu.sync_copy(zeros_v, acc.at[pl.ds(r0, ZR)])
    plsc.subcore_barrier()

    def chunkgrp(i, carry):
        g0 = i * NIDX
        for b in range(NIDX):
            g = g0 + b
            rb = b % NROW
            pltpu.make_async_copy(h_hbm.at[src_s[b]], rows[rb],
                                  gsem[rb]).wait()

            @pl.when(g + NIDX < NCHUNK)
            def _():
                _idx_load(edge_hbm, pad_hbm, 0, w, g + NIDX, src_s[b],
                          isem[b])

            pltpu.make_async_copy(edge_hbm.at[pl.ds(0, K)], dst_s[b],
                                  dsem[b]).wait()
            pltpu.sync_copy(rows[rb], acc.at[dst_s[b]], add=True)

            @pl.when(g + NIDX < NCHUNK)
            def _():
                _idx_load(edge_hbm, pad_hbm, 1, w, g + NIDX, dst_s[b],
                          dsem[b])

            @pl.when(g + NROW < NCHUNK)
            def _():
                tn = (b + NROW) % NIDX
                pltpu.make_async_copy(edge_hbm.at[pl.ds(0, K)],
                                      src_s[tn], isem[tn]).wait()
                pltpu.async_copy(h_hbm.at[src_s[tn]], rows[rb], gsem[rb])
        return carry

    lax.fori_loop(0, NCHUNK // NIDX, chunkgrp, 0)
    plsc.subcore_barrier()

    for j in range(ROWS_PER_SUB // K):
        r0 = s * ROWS_PER_SUB + j * K
        pltpu.sync_copy(acc.at[pl.ds(r0, K)], out_hbm.at[c, pl.ds(r0, K)])


def _make_segsum():
    mesh = plsc.VectorSubcoreMesh(core_axis_name="c", subcore_axis_name="s",
                                  num_cores=NC, num_subcores=NS)
    return pl.kernel(
        _segsum_body,
        out_type=jax.ShapeDtypeStruct((NC, ACC_ROWS, D), jnp.float32),
        mesh=mesh,
        scratch_types=[
            pltpu.VMEM((K,), jnp.int32),
            pltpu.VMEM((K,), jnp.int32),
            pltpu.VMEM((K,), jnp.int32),
            pltpu.VMEM((K,), jnp.int32),
            pltpu.VMEM((K,), jnp.int32),
            pltpu.VMEM((K,), jnp.int32),
            pltpu.VMEM((K,), jnp.int32),
            pltpu.VMEM((K,), jnp.int32),
            pltpu.VMEM((K, D), jnp.float32),
            pltpu.VMEM((K, D), jnp.float32),
            pltpu.VMEM((ZR, D), jnp.float32),
            pltpu.VMEM_SHARED((ACC_ROWS, D), jnp.float32),
            pltpu.SemaphoreType.DMA,
            pltpu.SemaphoreType.DMA,
            pltpu.SemaphoreType.DMA,
            pltpu.SemaphoreType.DMA,
            pltpu.SemaphoreType.DMA,
            pltpu.SemaphoreType.DMA,
            pltpu.SemaphoreType.DMA,
            pltpu.SemaphoreType.DMA,
            pltpu.SemaphoreType.DMA,
            pltpu.SemaphoreType.DMA,
        ],
    )


def _dense1_body(p_ref, x_ref, wrel_ref, wroot_ref, b_ref, o_ref):
    agg = p_ref[0] + p_ref[1]
    acc = jnp.dot(agg, wrel_ref[...], preferred_element_type=jnp.float32,
                  precision=lax.Precision.DEFAULT)
    acc += jnp.dot(x_ref[...], wroot_ref[...], preferred_element_type=jnp.float32,
                   precision=lax.Precision.DEFAULT)
    acc += b_ref[...]
    o_ref[...] = jnp.maximum(acc, 0.0)


def _dense1(p, x, W_rel, W_root, b):
    return pl.pallas_call(
        _dense1_body,
        grid=(NBLK,),
        in_specs=[
            pl.BlockSpec((NC, MB, D), lambda i: (0, i, 0)),
            pl.BlockSpec((MB, D), lambda i: (i, 0)),
            pl.BlockSpec((D, H), lambda i: (0, 0)),
            pl.BlockSpec((D, H), lambda i: (0, 0)),
            pl.BlockSpec((1, H), lambda i: (0, 0)),
        ],
        out_specs=pl.BlockSpec((MB, H), lambda i: (i, 0)),
        out_shape=jax.ShapeDtypeStruct((N, H), jnp.float32),
    )(p, x, W_rel, W_root, b.reshape(1, H))


def _dense2_body(p_ref, h_ref, wrel_ref, wroot_ref, b_ref, wc_ref, bc_ref, o_ref):
    agg = p_ref[0] + p_ref[1]
    t = jnp.dot(agg, wrel_ref[...], preferred_element_type=jnp.float32,
                precision=lax.Precision.DEFAULT)
    t += jnp.dot(h_ref[...], wroot_ref[...], preferred_element_type=jnp.float32,
                 precision=lax.Precision.DEFAULT)
    t += b_ref[...]
    o_ref[...] = jnp.dot(t, wc_ref[...], preferred_element_type=jnp.float32,
                         precision=lax.Precision.DEFAULT) + bc_ref[...]


def _dense2(p, h, W_rel, W_root, b, Wc, bc):
    return pl.pallas_call(
        _dense2_body,
        grid=(NBLK,),
        in_specs=[
            pl.BlockSpec((NC, MB, H), lambda i: (0, i, 0)),
            pl.BlockSpec((MB, H), lambda i: (i, 0)),
            pl.BlockSpec((H, H), lambda i: (0, 0)),
            pl.BlockSpec((H, H), lambda i: (0, 0)),
            pl.BlockSpec((1, H), lambda i: (0, 0)),
            pl.BlockSpec((H, C), lambda i: (0, 0)),
            pl.BlockSpec((1, C), lambda i: (0, 0)),
        ],
        out_specs=pl.BlockSpec((MB, C), lambda i: (i, 0)),
        out_shape=jax.ShapeDtypeStruct((N, C), jnp.float32),
    )(p, h, W_rel, W_root, b.reshape(1, H), Wc, bc.reshape(1, C))


_PAD_IDX = _np.arange(NW * 2 * K).reshape(NW * 2, K)
_PAD_NP = _np.concatenate([(_PAD_IDX * 41) % N,
                           N + _PAD_IDX % (ACC_ROWS - N)]).astype(_np.int32)


def kernel(x, edge_index, W1_rel, b1_rel, W1_root, W2_rel, b2_rel, W2_root, Wc, bc):
    pad_blk = jnp.asarray(_PAD_NP)
    edge_flat = edge_index.reshape(2 * E)
    segsum = _make_segsum()
    p1 = segsum(x, edge_flat, pad_blk)
    h1 = _dense1(p1, x, W1_rel, W1_root, b1_rel)
    p2 = segsum(h1, edge_flat, pad_blk)
    return _dense2(p2, h1, W2_rel, W2_root, b2_rel, Wc, bc)
